# baseline (speedup 1.0000x reference)
import functools

import jax
import jax.numpy as jnp
from jax import lax
from jax.experimental import pallas as pl
from jax.experimental.pallas import tpu as pltpu
from jax.experimental.pallas import tpu_sc as plsc

B = 2
NQ = 1600
FEAT = 2304
QT = 320
KT = 320
NKT = NQ // KT


def _topk_kernel(k_ref, q_ref, vals_ref, idx_ref, m_scr, i_scr):
    kt = pl.program_id(2)
    kmat = k_ref[0]
    qmat = q_ref[0]
    kn = jnp.sum(kmat * kmat, axis=1, keepdims=True)
    kmat = kmat / jnp.maximum(jnp.sqrt(kn), 1e-12)
    qn = jnp.sum(qmat * qmat, axis=1, keepdims=True)
    qmat = qmat / jnp.maximum(jnp.sqrt(qn), 1e-12)
    rel = lax.dot_general(kmat, qmat, (((1,), (1,)), ((), ())),
                          preferred_element_type=jnp.float32)
    tmax = jnp.max(rel, axis=0)
    rows = lax.broadcasted_iota(jnp.int32, rel.shape, 0) + kt * KT
    targ = jnp.min(jnp.where(rel == tmax[None, :], rows, 2**30), axis=0)

    @pl.when(kt == 0)
    def _():
        m_scr[0, :] = tmax
        i_scr[0, :] = targ

    @pl.when(kt > 0)
    def _():
        better = tmax > m_scr[0, :]
        m_scr[0, :] = jnp.where(better, tmax, m_scr[0, :])
        i_scr[0, :] = jnp.where(better, targ, i_scr[0, :])

    @pl.when(kt == pl.num_programs(2) - 1)
    def _():
        vals_ref[0, 0, :] = m_scr[0, :]
        idx_ref[0, 0, :] = i_scr[0, :]


def _top1(kmat, qmat):
    nqt = NQ // QT
    vals, idx = pl.pallas_call(
        _topk_kernel,
        grid=(B, nqt, NKT),
        in_specs=[
            pl.BlockSpec((1, KT, FEAT), lambda b, qt, kt: (b, kt, 0)),
            pl.BlockSpec((1, QT, FEAT), lambda b, qt, kt: (b, qt, 0)),
        ],
        out_specs=[
            pl.BlockSpec((1, 1, QT), lambda b, qt, kt: (b * nqt + qt, 0, 0)),
            pl.BlockSpec((1, 1, QT), lambda b, qt, kt: (b * nqt + qt, 0, 0)),
        ],
        out_shape=[
            jax.ShapeDtypeStruct((B * nqt, 1, QT), jnp.float32),
            jax.ShapeDtypeStruct((B * nqt, 1, QT), jnp.int32),
        ],
        scratch_shapes=[
            pltpu.VMEM((1, QT), jnp.float32),
            pltpu.VMEM((1, QT), jnp.int32),
        ],
    )(kmat, qmat)
    return vals.reshape(B, NQ), idx.reshape(B, NQ)


def _shr(v, n):
    return lax.shift_right_logical(v, n) if n else v


def _div_pow2_times5(v, k):
    t = _shr(v, k)
    return _shr(t * 52429, 18)


_DIV_SHIFT = {40: 3, 80: 4, 160: 5}


def _make_level_kernel(s, C, chunk):
    W = 40 * s
    npx = W * W
    P = 42 * s
    R = P * P
    log2s = {1: 0, 2: 1, 4: 2}[s]
    cpb = npx // chunk
    groups = chunk // 16
    mesh = plsc.VectorSubcoreMesh(core_axis_name="c", subcore_axis_name="s")

    cparams = {"needs_layout_passes": False}
    if C * 4 % 512:
        cparams["use_tc_tiling_on_sc"] = False

    @functools.partial(
        pl.kernel, mesh=mesh,
        compiler_params=pltpu.CompilerParams(**cparams),
        out_type=jax.ShapeDtypeStruct((B * npx, C), jnp.float32),
        scratch_types=[
            pltpu.VMEM((B, NQ), jnp.int32),
            pltpu.VMEM((9 * chunk,), jnp.int32),
            pltpu.VMEM((9 * chunk,), jnp.int32),
            pltpu.VMEM((chunk,), jnp.float32),
            pltpu.VMEM((chunk,), jnp.float32),
            pltpu.VMEM((9 * chunk, C), jnp.float32),
            pltpu.VMEM((9 * chunk, C), jnp.float32),
            pltpu.VMEM((chunk, C), jnp.float32),
            pltpu.VMEM((chunk, C), jnp.float32),
            pltpu.SemaphoreType.DMA,
            pltpu.SemaphoreType.DMA,
            pltpu.SemaphoreType.DMA,
            pltpu.SemaphoreType.DMA,
        ],
    )
    def level_kernel(v_hbm, idx_hbm, out_hbm, idxv, idxb0, idxb1, invb0, invb1,
                     rows0, rows1, outb0, outb1, smg0, smg1, smo0, smo1):
        wid = lax.axis_index("s") * 2 + lax.axis_index("c")
        lane = lax.iota(jnp.int32, 16)
        bufs = ((idxb0, invb0, rows0, outb0, smg0, smo0),
                (idxb1, invb1, rows1, outb1, smg1, smo1))
        pltpu.sync_copy(idx_hbm, idxv)
        nj = _shr(B * cpb - wid + 31, 5)
        nj2 = _shr(nj + 1, 1)

        def fire(j, ib, vb, rb, sm):
            ci = wid + 32 * j
            bsc = jnp.where(ci >= cpb, 1, 0)
            pbase = ci * chunk - bsc * npx
            bv = jnp.full((16,), 0, jnp.int32) + bsc
            for g in range(groups):
                pvec = pbase + g * 16 + lane
                y = _div_pow2_times5(pvec, _DIV_SHIFT[W])
                x = pvec - y * W
                yps = y + s
                xps = x + s
                fy = _shr(yps, log2s)
                ry = yps - lax.shift_left(fy, log2s)
                fx = _shr(xps, log2s)
                rx = xps - lax.shift_left(fx, log2s)
                cy = jnp.where(fy == 1, 2, 3) - jnp.where(fy == 40, 1, 0)
                cx = jnp.where(fx == 1, 2, 3) - jnp.where(fx == 40, 1, 0)
                inv = 1.0 / (cy * cx).astype(jnp.float32)
                vb[pl.ds(g * 16, 16)] = inv
                for t in range(9):
                    ti, tj = t // 3, t % 3
                    qy = fy - ti
                    qx = fx - tj
                    valid = (qy >= 0) & (qy < 40) & (qx >= 0) & (qx < 40)
                    q = jnp.where(valid, qy * 40 + qx, 0)
                    kidx = plsc.load_gather(idxv, [bv, q])
                    ky = _shr(kidx * 6554, 18)
                    kx = kidx - ky * 40
                    row = (s * ky + ry + ti * s) * P + (s * kx + rx + tj * s)
                    row = jnp.where(valid, row, 0) + bsc * R
                    ib[pl.ds(t * chunk + g * 16, 16)] = row
            pltpu.async_copy(v_hbm.at[ib], rb, sm)

        def drain(j, ib, vb, rb, ob, smg, smo):
            pltpu.make_async_copy(v_hbm.at[ib], rb, smg).wait()
            ci = wid + 32 * j

            @pl.when(j >= 2)
            def _():
                pltpu.make_async_copy(ob, out_hbm.at[pl.ds(0, chunk)], smo).wait()

            def px_body(pr, carry2, vb=vb, rb=rb, ob=ob):
                inv = plsc.load_gather(vb, [jnp.full((16,), 0, jnp.int32) + pr])
                for cb in range(C // 16):
                    acc = rb[pr, pl.ds(cb * 16, 16)]
                    for t in range(1, 9):
                        acc = acc + rb[t * chunk + pr, pl.ds(cb * 16, 16)]
                    ob[pr, pl.ds(cb * 16, 16)] = acc * inv
                return carry2

            lax.fori_loop(0, chunk, px_body, 0, unroll=2)
            pltpu.async_copy(ob, out_hbm.at[pl.ds(ci * chunk, chunk)], smo)

        def pair_body(j2, carry):
            for p in (0, 1):
                j = 2 * j2 + p
                ib, vb, rb, ob, smg, smo = bufs[p]

                @pl.when(j < nj)
                def _(j=j, ib=ib, vb=vb, rb=rb, smg=smg):
                    fire(j, ib, vb, rb, smg)

            for p in (0, 1):
                j = 2 * j2 + p
                ib, vb, rb, ob, smg, smo = bufs[p]

                @pl.when(j < nj)
                def _(j=j, ib=ib, vb=vb, rb=rb, ob=ob, smg=smg, smo=smo):
                    drain(j, ib, vb, rb, ob, smg, smo)

            return carry

        lax.fori_loop(0, nj2, pair_body, 0)
        for p in (0, 1):
            ib, vb, rb, ob, smg, smo = bufs[p]
            nfired = _shr(nj + 1 - p, 1)

            @pl.when(nfired > 0)
            def _(ob=ob, smo=smo):
                pltpu.make_async_copy(ob, out_hbm.at[pl.ds(0, chunk)], smo).wait()

    return level_kernel


def _pad_rows(cl, s):
    b, c, h, w = cl.shape
    p = jnp.pad(cl, ((0, 0), (0, 0), (s, s), (s, s)))
    return p.transpose(0, 2, 3, 1).reshape(b * (h + 2 * s) * (w + 2 * s), c)


def _unfold9(x):
    bb, c, h, w = x.shape
    xp = jnp.pad(x, ((0, 0), (0, 0), (1, 1), (1, 1)))
    cols = [xp[:, :, i:i + h, j:j + w] for i in range(3) for j in range(3)]
    out = jnp.stack(cols, axis=2)
    return out.reshape(bb, c * 9, h * w)


def kernel(dh_img_lv3, dh_ref_lv3, cl_ref_lv1, cl_ref_lv2, cl_ref_lv3):
    _, _, h, w = dh_img_lv3.shape
    qmat = jnp.swapaxes(_unfold9(dh_img_lv3), 1, 2)
    kmat = jnp.swapaxes(_unfold9(dh_ref_lv3), 1, 2)

    vals, idx = _top1(kmat, qmat)
    s_out = vals.reshape(B, 1, h, w)

    outs = []
    for cl, s, c, chunk in ((cl_ref_lv3, 1, 256, 16),
                            (cl_ref_lv2, 2, 128, 48),
                            (cl_ref_lv1, 4, 64, 80)):
        v = _pad_rows(cl, s)
        lk = _make_level_kernel(s, c, chunk)
        o = lk(v, idx)
        hw = 40 * s
        outs.append(o.reshape(B, hw, hw, c).transpose(0, 3, 1, 2))

    t3, t2, t1 = outs
    return (s_out, t3, t2, t1)


# speedup vs baseline: 1.0538x; 1.0538x over previous
import functools

import jax
import jax.numpy as jnp
from jax import lax
from jax.experimental import pallas as pl
from jax.experimental.pallas import tpu as pltpu
from jax.experimental.pallas import tpu_sc as plsc

B = 2
NQ = 1600
FEAT = 2304
QT = 320
KT = 320
NKT = NQ // KT


def _topk_kernel(k_ref, q_ref, vals_ref, idx_ref, m_scr, i_scr):
    kt = pl.program_id(2)
    kmat = k_ref[0]
    qmat = q_ref[0]
    kn = jnp.sum(kmat * kmat, axis=1, keepdims=True)
    kmat = kmat / jnp.maximum(jnp.sqrt(kn), 1e-12)
    qn = jnp.sum(qmat * qmat, axis=1, keepdims=True)
    qmat = qmat / jnp.maximum(jnp.sqrt(qn), 1e-12)
    rel = lax.dot_general(kmat, qmat, (((1,), (1,)), ((), ())),
                          preferred_element_type=jnp.float32)
    tmax = jnp.max(rel, axis=0)
    rows = lax.broadcasted_iota(jnp.int32, rel.shape, 0) + kt * KT
    targ = jnp.min(jnp.where(rel == tmax[None, :], rows, 2**30), axis=0)

    @pl.when(kt == 0)
    def _():
        m_scr[0, :] = tmax
        i_scr[0, :] = targ

    @pl.when(kt > 0)
    def _():
        better = tmax > m_scr[0, :]
        m_scr[0, :] = jnp.where(better, tmax, m_scr[0, :])
        i_scr[0, :] = jnp.where(better, targ, i_scr[0, :])

    @pl.when(kt == pl.num_programs(2) - 1)
    def _():
        vals_ref[0, 0, :] = m_scr[0, :]
        idx_ref[0, 0, :] = i_scr[0, :]


def _top1(kmat, qmat):
    nqt = NQ // QT
    vals, idx = pl.pallas_call(
        _topk_kernel,
        grid=(B, nqt, NKT),
        in_specs=[
            pl.BlockSpec((1, KT, FEAT), lambda b, qt, kt: (b, kt, 0)),
            pl.BlockSpec((1, QT, FEAT), lambda b, qt, kt: (b, qt, 0)),
        ],
        out_specs=[
            pl.BlockSpec((1, 1, QT), lambda b, qt, kt: (b * nqt + qt, 0, 0)),
            pl.BlockSpec((1, 1, QT), lambda b, qt, kt: (b * nqt + qt, 0, 0)),
        ],
        out_shape=[
            jax.ShapeDtypeStruct((B * nqt, 1, QT), jnp.float32),
            jax.ShapeDtypeStruct((B * nqt, 1, QT), jnp.int32),
        ],
        scratch_shapes=[
            pltpu.VMEM((1, QT), jnp.float32),
            pltpu.VMEM((1, QT), jnp.int32),
        ],
    )(kmat, qmat)
    return vals.reshape(B, NQ), idx.reshape(B, NQ)


def _shr(v, n):
    return lax.shift_right_logical(v, n) if n else v


def _div_pow2_times5(v, k):
    t = _shr(v, k)
    return _shr(t * 52429, 18)


_DIV_SHIFT = {40: 3, 80: 4, 160: 5}


def _make_level_kernel(s, C, chunk):
    W = 40 * s
    npx = W * W
    P = 42 * s
    R = P * P
    log2s = {1: 0, 2: 1, 4: 2}[s]
    cpb = npx // chunk
    groups = chunk // 16
    mesh = plsc.VectorSubcoreMesh(core_axis_name="c", subcore_axis_name="s")

    cparams = {"needs_layout_passes": False}
    if C * 4 % 512:
        cparams["use_tc_tiling_on_sc"] = False

    @functools.partial(
        pl.kernel, mesh=mesh,
        compiler_params=pltpu.CompilerParams(**cparams),
        out_type=jax.ShapeDtypeStruct((B * npx, C), jnp.float32),
        scratch_types=[
            pltpu.VMEM((B, NQ), jnp.int32),
            pltpu.VMEM((9 * chunk,), jnp.int32),
            pltpu.VMEM((9 * chunk,), jnp.int32),
            pltpu.VMEM((chunk,), jnp.float32),
            pltpu.VMEM((chunk,), jnp.float32),
            pltpu.VMEM((9 * chunk, C), jnp.float32),
            pltpu.VMEM((9 * chunk, C), jnp.float32),
            pltpu.VMEM((chunk, C), jnp.float32),
            pltpu.VMEM((chunk, C), jnp.float32),
            pltpu.SemaphoreType.DMA,
            pltpu.SemaphoreType.DMA,
            pltpu.SemaphoreType.DMA,
            pltpu.SemaphoreType.DMA,
        ],
    )
    def level_kernel(v_hbm, idx_hbm, out_hbm, idxv, idxb0, idxb1, invb0, invb1,
                     rows0, rows1, outb0, outb1, smg0, smg1, smo0, smo1):
        wid = lax.axis_index("s") * 2 + lax.axis_index("c")
        lane = lax.iota(jnp.int32, 16)
        bufs = ((idxb0, invb0, rows0, outb0, smg0, smo0),
                (idxb1, invb1, rows1, outb1, smg1, smo1))
        pltpu.sync_copy(idx_hbm, idxv)
        nj = _shr(B * cpb - wid + 31, 5)
        nj2 = _shr(nj + 1, 1)

        def fire(j, ib, vb, rb, sm):
            ci = wid + 32 * j
            bsc = jnp.where(ci >= cpb, 1, 0)
            pbase = ci * chunk - bsc * npx
            bv = jnp.full((16,), 0, jnp.int32) + bsc
            for g in range(groups):
                pvec = pbase + g * 16 + lane
                y = _div_pow2_times5(pvec, _DIV_SHIFT[W])
                x = pvec - y * W
                yps = y + s
                xps = x + s
                fy = _shr(yps, log2s)
                ry = yps - lax.shift_left(fy, log2s)
                fx = _shr(xps, log2s)
                rx = xps - lax.shift_left(fx, log2s)
                cy = jnp.where(fy == 1, 2, 3) - jnp.where(fy == 40, 1, 0)
                cx = jnp.where(fx == 1, 2, 3) - jnp.where(fx == 40, 1, 0)
                inv = 1.0 / (cy * cx).astype(jnp.float32)
                vb[pl.ds(g * 16, 16)] = inv
                for t in range(9):
                    ti, tj = t // 3, t % 3
                    qy = fy - ti
                    qx = fx - tj
                    valid = (qy >= 0) & (qy < 40) & (qx >= 0) & (qx < 40)
                    q = jnp.where(valid, qy * 40 + qx, 0)
                    kidx = plsc.load_gather(idxv, [bv, q])
                    ky = _shr(kidx * 6554, 18)
                    kx = kidx - ky * 40
                    row = (s * ky + ry + ti * s) * P + (s * kx + rx + tj * s)
                    row = jnp.where(valid, row, 0) + bsc * R
                    ib[pl.ds(t * chunk + g * 16, 16)] = row
            pltpu.async_copy(v_hbm.at[ib], rb, sm)

        def drain(j, ib, vb, rb, ob, smg, smo):
            pltpu.make_async_copy(v_hbm.at[ib], rb, smg).wait()
            ci = wid + 32 * j

            @pl.when(j >= 2)
            def _():
                pltpu.make_async_copy(ob, out_hbm.at[pl.ds(0, chunk)], smo).wait()

            def px_body(pr, carry2, vb=vb, rb=rb, ob=ob):
                inv = plsc.load_gather(vb, [jnp.full((16,), 0, jnp.int32) + pr])
                for cb in range(C // 16):
                    acc = rb[pr, pl.ds(cb * 16, 16)]
                    for t in range(1, 9):
                        acc = acc + rb[t * chunk + pr, pl.ds(cb * 16, 16)]
                    ob[pr, pl.ds(cb * 16, 16)] = acc * inv
                return carry2

            lax.fori_loop(0, chunk, px_body, 0, unroll=2)
            pltpu.async_copy(ob, out_hbm.at[pl.ds(ci * chunk, chunk)], smo)

        def pair_body(j2, carry):
            for p in (0, 1):
                j = 2 * j2 + p
                ib, vb, rb, ob, smg, smo = bufs[p]

                @pl.when(j < nj)
                def _(j=j, ib=ib, vb=vb, rb=rb, smg=smg):
                    fire(j, ib, vb, rb, smg)

            for p in (0, 1):
                j = 2 * j2 + p
                ib, vb, rb, ob, smg, smo = bufs[p]

                @pl.when(j < nj)
                def _(j=j, ib=ib, vb=vb, rb=rb, ob=ob, smg=smg, smo=smo):
                    drain(j, ib, vb, rb, ob, smg, smo)

            return carry

        lax.fori_loop(0, nj2, pair_body, 0)
        for p in (0, 1):
            ib, vb, rb, ob, smg, smo = bufs[p]
            nfired = _shr(nj + 1 - p, 1)

            @pl.when(nfired > 0)
            def _(ob=ob, smo=smo):
                pltpu.make_async_copy(ob, out_hbm.at[pl.ds(0, chunk)], smo).wait()

    return level_kernel


def _pad_rows(cl, s):
    b, c, h, w = cl.shape
    p = jnp.pad(cl, ((0, 0), (0, 0), (s, s), (s, s)))
    return p.transpose(0, 2, 3, 1).reshape(b * (h + 2 * s) * (w + 2 * s), c)


def _unfold9(x):
    bb, c, h, w = x.shape
    xp = jnp.pad(x, ((0, 0), (0, 0), (1, 1), (1, 1)))
    cols = [xp[:, :, i:i + h, j:j + w] for i in range(3) for j in range(3)]
    out = jnp.stack(cols, axis=2)
    return out.reshape(bb, c * 9, h * w)


def kernel(dh_img_lv3, dh_ref_lv3, cl_ref_lv1, cl_ref_lv2, cl_ref_lv3):
    _, _, h, w = dh_img_lv3.shape
    qmat = jnp.swapaxes(_unfold9(dh_img_lv3), 1, 2)
    kmat = jnp.swapaxes(_unfold9(dh_ref_lv3), 1, 2)

    vals, idx = _top1(kmat, qmat)
    s_out = vals.reshape(B, 1, h, w)

    outs = []
    for cl, s, c, chunk in ((cl_ref_lv3, 1, 256, 16),
                            (cl_ref_lv2, 2, 128, 32),
                            (cl_ref_lv1, 4, 64, 64)):
        v = _pad_rows(cl, s)
        lk = _make_level_kernel(s, c, chunk)
        o = lk(v, idx)
        hw = 40 * s
        outs.append(o.reshape(B, hw, hw, c).transpose(0, 3, 1, 2))

    t3, t2, t1 = outs
    return (s_out, t3, t2, t1)
